# baseline (speedup 1.0000x reference)
import functools

import jax
import jax.numpy as jnp
from jax import lax
from jax.experimental import pallas as pl
from jax.experimental.pallas import tpu as pltpu
from jax.experimental.pallas import tpu_sc as plsc

HIST = 200
EMB = 64
OUT_COLS = EMB + 1
ROWS_PER_WORKER = 8
NUM_WORKERS = HIST // ROWS_PER_WORKER
BLOCK = ROWS_PER_WORKER * OUT_COLS
TILE = 128


def _make_sc_lookup():
    mesh = plsc.VectorSubcoreMesh(core_axis_name="c", subcore_axis_name="s")

    @functools.partial(
        pl.kernel,
        mesh=mesh,
        compiler_params=pltpu.CompilerParams(needs_layout_passes=False),
        out_type=jax.ShapeDtypeStruct((HIST * OUT_COLS,), jnp.float32),
        scratch_types=[
            pltpu.VMEM((HIST,), jnp.int32),
            pltpu.VMEM((HIST,), jnp.int32),
            pltpu.VMEM((ROWS_PER_WORKER, EMB, TILE), jnp.float32),
            pltpu.VMEM((BLOCK + 16,), jnp.float32),
            pltpu.SemaphoreType.DMA,
        ],
    )
    def sc_lookup(st_hbm, tableT_hbm, out_hbm, ids_v, marks_v, bufs_v,
                  out_v, sem):
        wid = lax.axis_index("s") * 2 + lax.axis_index("c")

        @pl.when(wid < NUM_WORKERS)
        def _():
            base = wid * ROWS_PER_WORKER
            c_idx = pltpu.async_copy(st_hbm.at[0], ids_v, sem)
            c_marks = pltpu.async_copy(st_hbm.at[1], marks_v, sem)
            c_idx.wait()
            lanes0 = lax.iota(jnp.int32, 16)
            sel = jnp.where(lanes0 < ROWS_PER_WORKER, base + lanes0, 0)
            idxs = plsc.load_gather(ids_v, [sel])
            copies = []
            for i in range(ROWS_PER_WORKER):
                tile_base = pl.multiple_of((idxs[i] // TILE) * TILE, TILE)
                copies.append(pltpu.async_copy(
                    tableT_hbm.at[:, pl.ds(tile_base, TILE)],
                    bufs_v.at[i], sem))
            c_marks.wait()
            for c in copies:
                c.wait()
            lane = lanes0
            marks_f = plsc.load_gather(marks_v, [sel]).astype(jnp.float32)
            for i in range(ROWS_PER_WORKER - 1, -1, -1):
                rm = jnp.full((16,), idxs[i] % TILE, jnp.int32)
                for j in range(EMB // 16):
                    vals = plsc.load_gather(bufs_v.at[i],
                                            [lane + 16 * j, rm])
                    out_v[pl.ds(i * OUT_COLS + 16 * j, 16)] = vals
                win = out_v[pl.ds(i * OUT_COLS + EMB, 16)]
                out_v[pl.ds(i * OUT_COLS + EMB, 16)] = jnp.where(
                    lane == 0, marks_f[i], win)
            pltpu.sync_copy(out_v.at[pl.ds(0, BLOCK)],
                            out_hbm.at[pl.ds(base * OUT_COLS, BLOCK)])

    return sc_lookup


_sc_lookup = _make_sc_lookup()


def kernel(st, task_emb, null_mark_emb):
    del null_mark_emb
    out_flat = _sc_lookup(st.astype(jnp.int32), task_emb.T)
    return jnp.reshape(out_flat, (1, HIST, OUT_COLS))


# speedup vs baseline: 1.0279x; 1.0279x over previous
import functools

import jax
import jax.numpy as jnp
from jax import lax
from jax.experimental import pallas as pl
from jax.experimental.pallas import tpu as pltpu
from jax.experimental.pallas import tpu_sc as plsc

HIST = 200
EMB = 64
OUT_COLS = EMB + 1
ROWS_PER_WORKER = 8
NUM_WORKERS = HIST // ROWS_PER_WORKER
BLOCK = ROWS_PER_WORKER * OUT_COLS
TILE = 128


def _make_sc_lookup():
    mesh = plsc.VectorSubcoreMesh(core_axis_name="c", subcore_axis_name="s")

    @functools.partial(
        pl.kernel,
        mesh=mesh,
        compiler_params=pltpu.CompilerParams(needs_layout_passes=False),
        out_type=jax.ShapeDtypeStruct((HIST * OUT_COLS,), jnp.float32),
        scratch_types=[
            pltpu.VMEM((16,), jnp.int32),
            pltpu.VMEM((16,), jnp.int32),
            pltpu.VMEM((ROWS_PER_WORKER, EMB, TILE), jnp.float32),
            pltpu.VMEM((BLOCK + 16,), jnp.float32),
            pltpu.SemaphoreType.DMA,
        ],
    )
    def sc_lookup(st_flat_hbm, tableT_hbm, out_hbm, idx_v, marks_v, bufs_v,
                  out_v, sem):
        wid = lax.axis_index("s") * 2 + lax.axis_index("c")

        @pl.when(wid < NUM_WORKERS)
        def _():
            base = wid * ROWS_PER_WORKER
            c_idx = pltpu.async_copy(
                st_flat_hbm.at[pl.ds(base, ROWS_PER_WORKER)],
                idx_v.at[pl.ds(0, ROWS_PER_WORKER)], sem)
            c_marks = pltpu.async_copy(
                st_flat_hbm.at[pl.ds(HIST + base, ROWS_PER_WORKER)],
                marks_v.at[pl.ds(0, ROWS_PER_WORKER)], sem)
            c_idx.wait()
            idxs = idx_v[...]
            copies = []
            for i in range(ROWS_PER_WORKER):
                tile_base = pl.multiple_of((idxs[i] // TILE) * TILE, TILE)
                copies.append(pltpu.async_copy(
                    tableT_hbm.at[:, pl.ds(tile_base, TILE)],
                    bufs_v.at[i], sem))
            c_marks.wait()
            for c in copies:
                c.wait()
            lane = lax.iota(jnp.int32, 16)
            marks_f = marks_v[...].astype(jnp.float32)
            for i in range(ROWS_PER_WORKER - 1, -1, -1):
                rm = jnp.full((16,), idxs[i] % TILE, jnp.int32)
                for j in range(EMB // 16):
                    vals = plsc.load_gather(bufs_v.at[i],
                                            [lane + 16 * j, rm])
                    out_v[pl.ds(i * OUT_COLS + 16 * j, 16)] = vals
                win = out_v[pl.ds(i * OUT_COLS + EMB, 16)]
                out_v[pl.ds(i * OUT_COLS + EMB, 16)] = jnp.where(
                    lane == 0, marks_f[i], win)
            pltpu.sync_copy(out_v.at[pl.ds(0, BLOCK)],
                            out_hbm.at[pl.ds(base * OUT_COLS, BLOCK)])

    return sc_lookup


_sc_lookup = _make_sc_lookup()


def kernel(st, task_emb, null_mark_emb):
    del null_mark_emb
    st_flat = jnp.reshape(st, (2 * HIST,)).astype(jnp.int32)
    out_flat = _sc_lookup(st_flat, task_emb.T)
    return jnp.reshape(out_flat, (1, HIST, OUT_COLS))


# speedup vs baseline: 1.0348x; 1.0067x over previous
import functools

import jax
import jax.numpy as jnp
from jax import lax
from jax.experimental import pallas as pl
from jax.experimental.pallas import tpu as pltpu
from jax.experimental.pallas import tpu_sc as plsc

HIST = 200
EMB = 64
OUT_COLS = EMB + 1
ROWS_PER_WORKER = 8
NUM_WORKERS = HIST // ROWS_PER_WORKER
BLOCK = ROWS_PER_WORKER * OUT_COLS
TILE = 128


def _make_sc_lookup():
    mesh = plsc.VectorSubcoreMesh(core_axis_name="c", subcore_axis_name="s")

    @functools.partial(
        pl.kernel,
        mesh=mesh,
        compiler_params=pltpu.CompilerParams(needs_layout_passes=False),
        out_type=jax.ShapeDtypeStruct((HIST * OUT_COLS,), jnp.float32),
        scratch_types=[
            pltpu.VMEM((16,), jnp.int32),
            pltpu.VMEM((16,), jnp.int32),
            pltpu.VMEM((ROWS_PER_WORKER, EMB, TILE), jnp.float32),
            pltpu.VMEM((BLOCK + 16,), jnp.float32),
            pltpu.SemaphoreType.DMA,
        ],
    )
    def sc_lookup(st_flat_hbm, tableT_hbm, out_hbm, idx_v, marks_v, bufs_v,
                  out_v, sem):
        wid = lax.axis_index("s") * 2 + lax.axis_index("c")

        @pl.when(wid < NUM_WORKERS)
        def _():
            base = wid * ROWS_PER_WORKER
            c_idx = pltpu.async_copy(
                st_flat_hbm.at[pl.ds(base, ROWS_PER_WORKER)],
                idx_v.at[pl.ds(0, ROWS_PER_WORKER)], sem)
            c_marks = pltpu.async_copy(
                st_flat_hbm.at[pl.ds(HIST + base, ROWS_PER_WORKER)],
                marks_v.at[pl.ds(0, ROWS_PER_WORKER)], sem)
            c_idx.wait()
            idxs = idx_v[...]
            copies = []
            for i in range(ROWS_PER_WORKER):
                tile_base = pl.multiple_of((idxs[i] // TILE) * TILE, TILE)
                copies.append(pltpu.async_copy(
                    tableT_hbm.at[:, pl.ds(tile_base, TILE)],
                    bufs_v.at[i], sem))
            c_marks.wait()
            lane = lax.iota(jnp.int32, 16)
            marks_f = marks_v[...].astype(jnp.float32)
            for i in range(ROWS_PER_WORKER):
                copies[i].wait()
                rm = jnp.full((16,), idxs[i] % TILE, jnp.int32)
                for j in range(EMB // 16):
                    vals = plsc.load_gather(bufs_v.at[i],
                                            [lane + 16 * j, rm])
                    out_v[pl.ds(i * OUT_COLS + 16 * j, 16)] = vals
                win = out_v[pl.ds(i * OUT_COLS + EMB, 16)]
                out_v[pl.ds(i * OUT_COLS + EMB, 16)] = jnp.where(
                    lane == 0, marks_f[i], win)
            pltpu.sync_copy(out_v.at[pl.ds(0, BLOCK)],
                            out_hbm.at[pl.ds(base * OUT_COLS, BLOCK)])

    return sc_lookup


_sc_lookup = _make_sc_lookup()


def kernel(st, task_emb, null_mark_emb):
    del null_mark_emb
    st_flat = jnp.reshape(st, (2 * HIST,)).astype(jnp.int32)
    out_flat = _sc_lookup(st_flat, task_emb.T)
    return jnp.reshape(out_flat, (1, HIST, OUT_COLS))
